# Initial kernel scaffold; baseline (speedup 1.0000x reference)
import jax
import jax.numpy as jnp
from jax.experimental import pallas as pl


def kernel(z, edge_index, proj_W, proj_b, W0, b0, W1, b1, W2, b2):
    raise NotImplementedError("write your pallas kernel here")


# speedup vs baseline: 8.1922x; 8.1922x over previous
import functools

import jax
import jax.numpy as jnp
from jax.experimental import pallas as pl
from jax.experimental.pallas import tpu as pltpu

N = 10000
E = 160000
LATENT = 128
FIRST = 32
B = 16
NPAD = 10240


def _mm_kernel(x_ref, y_ref, o_ref, acc_ref, *, nk):
    k = pl.program_id(2)

    @pl.when(k == 0)
    def _():
        acc_ref[...] = jnp.zeros_like(acc_ref)

    acc_ref[...] += jnp.dot(x_ref[...], y_ref[...],
                            preferred_element_type=jnp.float32)

    @pl.when(k == nk - 1)
    def _():
        o_ref[...] = acc_ref[...]


def _mm(x, y, bm, bn, bk):
    M, K = x.shape
    _, Nn = y.shape
    grid = (M // bm, Nn // bn, K // bk)
    return pl.pallas_call(
        functools.partial(_mm_kernel, nk=grid[2]),
        grid=grid,
        in_specs=[pl.BlockSpec((bm, bk), lambda i, j, k: (i, k)),
                  pl.BlockSpec((bk, bn), lambda i, j, k: (k, j))],
        out_specs=pl.BlockSpec((bm, bn), lambda i, j, k: (i, j)),
        out_shape=jax.ShapeDtypeStruct((M, Nn), jnp.float32),
        scratch_shapes=[pltpu.VMEM((bm, bn), jnp.float32)],
    )(x, y)


def _proj_kernel(z_ref, w_ref, b_ref, o_ref):
    o_ref[...] = (jnp.dot(z_ref[...], w_ref[...],
                          preferred_element_type=jnp.float32)
                  + b_ref[...])


def _proj(z, w, b, bn):
    _, NF = w.shape
    grid = (NF // bn,)
    return pl.pallas_call(
        _proj_kernel,
        grid=grid,
        in_specs=[pl.BlockSpec((B, LATENT), lambda j: (0, 0)),
                  pl.BlockSpec((LATENT, bn), lambda j: (0, j)),
                  pl.BlockSpec((1, bn), lambda j: (0, j))],
        out_specs=pl.BlockSpec((B, bn), lambda j: (0, j)),
        out_shape=jax.ShapeDtypeStruct((B, NF), jnp.float32),
    )(z, w, b)


def _combine_kernel(x_ref, s1_ref, s2_ref, w0_ref, w1_ref, w2_ref, b_ref,
                    o_ref, *, act):
    xb = x_ref[...]
    t2 = 2.0 * s2_ref[...] - xb
    acc = jnp.dot(xb, w0_ref[...], preferred_element_type=jnp.float32)
    acc += jnp.dot(s1_ref[...], w1_ref[...],
                   preferred_element_type=jnp.float32)
    acc += jnp.dot(t2, w2_ref[...], preferred_element_type=jnp.float32)
    acc += b_ref[...]
    if act:
        acc = jnp.where(acc > 0.0, acc, jnp.exp(acc) - 1.0)
    o_ref[...] = acc


def _combine(x2d, s1, s2, w0, w1, w2, b, act, bm):
    M, F = x2d.shape
    O = w0.shape[1]
    grid = (M // bm,)
    return pl.pallas_call(
        functools.partial(_combine_kernel, act=act),
        grid=grid,
        in_specs=[pl.BlockSpec((bm, F), lambda i: (i, 0)),
                  pl.BlockSpec((bm, F), lambda i: (i, 0)),
                  pl.BlockSpec((bm, F), lambda i: (i, 0)),
                  pl.BlockSpec((F, O), lambda i: (0, 0)),
                  pl.BlockSpec((F, O), lambda i: (0, 0)),
                  pl.BlockSpec((F, O), lambda i: (0, 0)),
                  pl.BlockSpec((1, O), lambda i: (0, 0))],
        out_specs=pl.BlockSpec((bm, O), lambda i: (i, 0)),
        out_shape=jax.ShapeDtypeStruct((M, O), jnp.float32),
    )(x2d, s1, s2, w0, w1, w2, b)


def kernel(z, edge_index, proj_W, proj_b, W0, b0, W1, b1, W2, b2):
    src = edge_index[0].astype(jnp.int32)
    dst = edge_index[1].astype(jnp.int32)
    ones = jnp.ones((E,), dtype=jnp.float32)
    deg_out = jnp.zeros((N,), jnp.float32).at[src].add(ones)
    deg_in = jnp.zeros((N,), jnp.float32).at[dst].add(ones)
    dinv_out = 1.0 / jnp.sqrt(jnp.maximum(deg_out, 1.0))
    dinv_in = 1.0 / jnp.sqrt(jnp.maximum(deg_in, 1.0))
    w = -(dinv_out[src] * dinv_in[dst])
    ahat = jnp.zeros((NPAD, NPAD), jnp.float32).at[dst, src].add(w)

    nf = N * FIRST
    x0 = _proj(z, proj_W, proj_b.reshape(1, -1),
               2560 if nf % 2560 == 0 else nf)
    x = x0.reshape(B, N, FIRST)

    for (W, b, act) in ((W0, b0, True), (W1, b1, True), (W2, b2, False)):
        Fin = x.shape[2]
        C = B * Fin
        xt = jnp.swapaxes(x, 0, 1).reshape(N, C)
        xt = jnp.pad(xt, ((0, NPAD - N), (0, 0)))
        s1t = _mm(ahat, xt, min(256, NPAD), min(C, 512), min(512, NPAD))
        s2t = _mm(ahat, s1t, min(256, NPAD), min(C, 512), min(512, NPAD))
        s1 = jnp.swapaxes(s1t[:N].reshape(N, B, Fin), 0, 1)
        s2 = jnp.swapaxes(s2t[:N].reshape(N, B, Fin), 0, 1)
        out = _combine(x.reshape(B * N, Fin), s1.reshape(B * N, Fin),
                       s2.reshape(B * N, Fin), W[0], W[1], W[2],
                       b.reshape(1, -1), act, 1000)
        x = out.reshape(B, N, out.shape[1])

    return x


# speedup vs baseline: 18.5697x; 2.2668x over previous
import functools

import jax
import jax.numpy as jnp
from jax import lax
from jax.experimental import pallas as pl
from jax.experimental.pallas import tpu as pltpu
from jax.experimental.pallas import tpu_sc as plsc

N = 10000
E = 160000
LATENT = 128
FIRST = 32
B = 16

NSC = 2
NSUB = 16
NPAD = 10240
RPS = NPAD // NSUB
EB = 128
CK = 128
NBLK = E // EB
HBLK = NBLK // NSC
IPS = -(-HBLK // NSUB)


def _make_lap(C):
    nck = C // CK
    mesh = plsc.VectorSubcoreMesh(core_axis_name="c", subcore_axis_name="s")

    @functools.partial(
        pl.kernel,
        mesh=mesh,
        out_type=jax.ShapeDtypeStruct((NSC, NPAD, C), jnp.float32),
        scratch_types=[
            pltpu.VMEM_SHARED((NPAD, CK), jnp.float32),
            pltpu.VMEM((EB, CK), jnp.float32),
            pltpu.VMEM((EB,), jnp.int32),
            pltpu.VMEM((EB,), jnp.int32),
            pltpu.SemaphoreType.DMA,
        ],
    )
    def lap(y_hbm, srcs_hbm, dsts_hbm, zeros_hbm, out_hbm,
            acc, gbuf, srcv, dstv, sem):
        c = lax.axis_index("c")
        s = lax.axis_index("s")

        for p in range(nck):
            pltpu.sync_copy(zeros_hbm, acc.at[pl.ds(s * RPS, RPS)])
            plsc.subcore_barrier()

            def blk(i, carry):
                kl = s + i * NSUB

                @pl.when(kl < HBLK)
                def _():
                    base = (c * HBLK + kl) * EB
                    pltpu.sync_copy(srcs_hbm.at[pl.ds(base, EB)], srcv)
                    pltpu.sync_copy(dsts_hbm.at[pl.ds(base, EB)], dstv)
                    pltpu.async_copy(
                        y_hbm.at[srcv, pl.ds(p * CK, CK)], gbuf, sem).wait()
                    pltpu.sync_copy(gbuf, acc.at[dstv], add=True)

                return carry

            lax.fori_loop(0, IPS, blk, 0)

            plsc.subcore_barrier()
            pltpu.sync_copy(
                acc.at[pl.ds(s * RPS, RPS)],
                out_hbm.at[c, pl.ds(s * RPS, RPS), pl.ds(p * CK, CK)])

    return lap


def _proj_kernel(z_ref, w_ref, b_ref, o_ref):
    o_ref[...] = (jnp.dot(z_ref[...], w_ref[...],
                          preferred_element_type=jnp.float32)
                  + b_ref[...])


def _proj(z, w, b, bn):
    NF = w.shape[1]
    grid = (NF // bn,)
    return pl.pallas_call(
        _proj_kernel,
        grid=grid,
        in_specs=[pl.BlockSpec((B, LATENT), lambda j: (0, 0)),
                  pl.BlockSpec((LATENT, bn), lambda j: (0, j)),
                  pl.BlockSpec((1, bn), lambda j: (0, j))],
        out_specs=pl.BlockSpec((B, bn), lambda j: (0, j)),
        out_shape=jax.ShapeDtypeStruct((B, NF), jnp.float32),
    )(z, w, b)


def _scale_kernel(a_ref, s_ref, o_ref):
    o_ref[...] = a_ref[...] * s_ref[...]


def _scale(a, s, bm=2000):
    M, F = a.shape
    grid = (M // bm,)
    row = lambda i: (i, 0)
    return pl.pallas_call(
        _scale_kernel,
        grid=grid,
        in_specs=[pl.BlockSpec((bm, F), row), pl.BlockSpec((bm, 1), row)],
        out_specs=pl.BlockSpec((bm, F), row),
        out_shape=jax.ShapeDtypeStruct((M, F), jnp.float32),
    )(a, s)


def _elu(x):
    return jnp.where(x > 0.0, x, jnp.exp(x) - 1.0)


def _combine_a_kernel(x_ref, sa_ref, sb_ref, din_ref, dout_ref,
                      w0_ref, w1_ref, w2_ref, b_ref, o_ref, y_ref):
    x = x_ref[...]
    din = din_ref[...]
    t1 = -din * sa_ref[...]
    t2 = -2.0 * din * sb_ref[...] - x
    acc = jnp.dot(x, w0_ref[...], preferred_element_type=jnp.float32)
    acc += jnp.dot(t1, w1_ref[...], preferred_element_type=jnp.float32)
    acc += jnp.dot(t2, w2_ref[...], preferred_element_type=jnp.float32)
    acc = _elu(acc + b_ref[...])
    o_ref[...] = acc
    y_ref[...] = acc * dout_ref[...]


def _combine_a(x, sa, sb, din, dout, W, b, bm):
    M, F = x.shape
    O = W.shape[2]
    grid = (M // bm,)
    full = lambda i: (0, 0)
    row = lambda i: (i, 0)
    return pl.pallas_call(
        _combine_a_kernel,
        grid=grid,
        in_specs=[pl.BlockSpec((bm, F), row), pl.BlockSpec((bm, F), row),
                  pl.BlockSpec((bm, F), row), pl.BlockSpec((bm, 1), row),
                  pl.BlockSpec((bm, 1), row),
                  pl.BlockSpec((F, O), full), pl.BlockSpec((F, O), full),
                  pl.BlockSpec((F, O), full), pl.BlockSpec((1, O), full)],
        out_specs=[pl.BlockSpec((bm, O), row), pl.BlockSpec((bm, O), row)],
        out_shape=[jax.ShapeDtypeStruct((M, O), jnp.float32),
                   jax.ShapeDtypeStruct((M, O), jnp.float32)],
    )(x, sa, sb, din, dout, W[0], W[1], W[2], b)


def _combine_b_kernel(x_ref, sa_ref, sb_ref, din_ref, dout_ref,
                      w0_ref, w1_ref, w2_ref, b_ref, wp1_ref, wp2_ref,
                      o_ref, yq_ref):
    x = x_ref[...]
    din = din_ref[...]
    t1 = -din * sa_ref[...]
    t2 = -2.0 * din * sb_ref[...] - x
    acc = jnp.dot(x, w0_ref[...], preferred_element_type=jnp.float32)
    acc += jnp.dot(t1, w1_ref[...], preferred_element_type=jnp.float32)
    acc += jnp.dot(t2, w2_ref[...], preferred_element_type=jnp.float32)
    acc = _elu(acc + b_ref[...])
    o_ref[...] = acc
    p1 = jnp.dot(acc, wp1_ref[...], preferred_element_type=jnp.float32)
    p2 = jnp.dot(acc, wp2_ref[...], preferred_element_type=jnp.float32)
    yq_ref[...] = jnp.concatenate([p1, p2], axis=1) * dout_ref[...]


def _combine_b(x, sa, sb, din, dout, W, b, wp1, wp2, bm):
    M, F = x.shape
    O = W.shape[2]
    O2 = 2 * wp1.shape[1]
    grid = (M // bm,)
    full = lambda i: (0, 0)
    row = lambda i: (i, 0)
    return pl.pallas_call(
        _combine_b_kernel,
        grid=grid,
        in_specs=[pl.BlockSpec((bm, F), row), pl.BlockSpec((bm, F), row),
                  pl.BlockSpec((bm, F), row), pl.BlockSpec((bm, 1), row),
                  pl.BlockSpec((bm, 1), row),
                  pl.BlockSpec((F, O), full), pl.BlockSpec((F, O), full),
                  pl.BlockSpec((F, O), full), pl.BlockSpec((1, O), full),
                  pl.BlockSpec((O, wp1.shape[1]), full),
                  pl.BlockSpec((O, wp2.shape[1]), full)],
        out_specs=[pl.BlockSpec((bm, O), row), pl.BlockSpec((bm, O2), row)],
        out_shape=[jax.ShapeDtypeStruct((M, O), jnp.float32),
                   jax.ShapeDtypeStruct((M, O2), jnp.float32)],
    )(x, sa, sb, din, dout, W[0], W[1], W[2], b, wp1, wp2)


def _final_kernel(x_ref, u1_ref, u3_ref, din_ref, wd_ref, b_ref, o_ref):
    din = din_ref[...]
    o_ref[...] = (jnp.dot(x_ref[...], wd_ref[...],
                          preferred_element_type=jnp.float32)
                  - din * u1_ref[...] - 2.0 * din * u3_ref[...] + b_ref[...])


def _final(x, u1, u3, din, wd, b, bm):
    M, F = x.shape
    O = wd.shape[1]
    grid = (M // bm,)
    full = lambda i: (0, 0)
    row = lambda i: (i, 0)
    return pl.pallas_call(
        _final_kernel,
        grid=grid,
        in_specs=[pl.BlockSpec((bm, F), row), pl.BlockSpec((bm, O), row),
                  pl.BlockSpec((bm, O), row), pl.BlockSpec((bm, 1), row),
                  pl.BlockSpec((F, O), full), pl.BlockSpec((1, O), full)],
        out_specs=pl.BlockSpec((bm, O), row),
        out_shape=jax.ShapeDtypeStruct((M, O), jnp.float32),
    )(x, u1, u3, din, wd, b)


def kernel(z, edge_index, proj_W, proj_b, W0, b0, W1, b1, W2, b2):
    src = edge_index[0].astype(jnp.int32)
    dst = edge_index[1].astype(jnp.int32)
    deg_out = jnp.zeros((N,), jnp.float32).at[src].add(1.0)
    deg_in = jnp.zeros((N,), jnp.float32).at[dst].add(1.0)
    dinv_out = lax.rsqrt(jnp.maximum(deg_out, 1.0))
    dinv_in = lax.rsqrt(jnp.maximum(deg_in, 1.0))
    ms = -(dinv_out * dinv_in)

    din_col = jnp.repeat(dinv_in, B)[:, None]
    dout_col = jnp.repeat(dinv_out, B)[:, None]
    ms_col = jnp.repeat(ms, B)[:, None]
    zbuf = jnp.zeros((RPS, CK), jnp.float32)

    lap512 = _make_lap(512)
    lap1024 = _make_lap(1024)
    lap256 = _make_lap(256)
    M = N * B

    x0 = _proj(z, proj_W, proj_b.reshape(1, -1), 2560)
    xt = jnp.swapaxes(x0.reshape(B, N, 32), 0, 1)
    x0r = xt.reshape(M, 32)
    y0 = _scale(x0r, dout_col)
    sa2 = lap512(y0.reshape(N, 512), src, dst, zbuf)
    sa = sa2[0] + sa2[1]
    san = sa[:N].reshape(M, 32)
    y1 = _scale(san, ms_col)
    sb2 = lap512(y1.reshape(N, 512), src, dst, zbuf)
    sb = sb2[0] + sb2[1]
    sbn = sb[:N].reshape(M, 32)
    out0, ynext = _combine_a(x0r, san, sbn, din_col, dout_col,
                             W0, b0.reshape(1, -1), 1000)

    sa12 = lap1024(ynext.reshape(N, 1024), src, dst, zbuf)
    sa1 = sa12[0] + sa12[1]
    sa1n = sa1[:N].reshape(M, 64)
    y11 = _scale(sa1n, ms_col)
    sb12 = lap1024(y11.reshape(N, 1024), src, dst, zbuf)
    sb1 = sb12[0] + sb12[1]
    sb1n = sb1[:N].reshape(M, 64)
    out1, yq = _combine_b(out0, sa1n, sb1n, din_col, dout_col,
                          W1, b1.reshape(1, -1), W2[1], W2[2], 1000)

    u122 = lap512(yq.reshape(N, 512), src, dst, zbuf)
    u12 = u122[0] + u122[1]
    u12n = u12[:N].reshape(M, 32)
    u1 = u12n[:, :16]
    u2 = u12n[:, 16:]
    yq3 = _scale(u2, ms_col)
    u3p2 = lap256(yq3.reshape(N, 256), src, dst, zbuf)
    u3p = u3p2[0] + u3p2[1]
    u3 = u3p[:N].reshape(M, 16)
    out2 = _final(out1, u1, u3, din_col, W2[0] - W2[2],
                  b2.reshape(1, -1), 1000)
    return jnp.swapaxes(out2.reshape(N, B, 16), 0, 1)


# speedup vs baseline: 21.6613x; 1.1665x over previous
import functools

import jax
import jax.numpy as jnp
from jax import lax
from jax.experimental import pallas as pl
from jax.experimental.pallas import tpu as pltpu
from jax.experimental.pallas import tpu_sc as plsc

N = 10000
E = 160000
LATENT = 128
FIRST = 32
B = 16

NSC = 2
NSUB = 16
NPAD = 10240
RPS = NPAD // NSUB
EB = 128
CK = 128
NBLK = E // EB
HBLK = NBLK // NSC
IPS = -(-HBLK // NSUB)


def _make_lap(C):
    nck = C // CK
    mesh = plsc.VectorSubcoreMesh(core_axis_name="c", subcore_axis_name="s")

    @functools.partial(
        pl.kernel,
        mesh=mesh,
        out_type=jax.ShapeDtypeStruct((NSC, NPAD, C), jnp.float32),
        scratch_types=[
            pltpu.VMEM_SHARED((NPAD, CK), jnp.float32),
            pltpu.VMEM((2, EB, CK), jnp.float32),
            pltpu.VMEM((2, EB), jnp.int32),
            pltpu.VMEM((2, EB), jnp.int32),
            pltpu.SemaphoreType.DMA,
            pltpu.SemaphoreType.DMA,
        ],
    )
    def lap(y_hbm, srcs_hbm, dsts_hbm, zeros_hbm, out_hbm,
            acc, gbuf, srcv, dstv, sem0, sem1):
        c = lax.axis_index("c")
        s = lax.axis_index("s")
        sems = (sem0, sem1)

        for p in range(nck):
            pltpu.sync_copy(zeros_hbm, acc.at[pl.ds(s * RPS, RPS)])
            plsc.subcore_barrier()

            def gcp(par):
                return pltpu.make_async_copy(
                    y_hbm.at[srcv.at[par], pl.ds(p * CK, CK)],
                    gbuf.at[par], sems[par])

            def start(i, par):
                kl = s + i * NSUB

                @pl.when(kl < HBLK)
                def _():
                    base = (c * HBLK + kl) * EB
                    pltpu.sync_copy(srcs_hbm.at[pl.ds(base, EB)],
                                    srcv.at[par])
                    pltpu.sync_copy(dsts_hbm.at[pl.ds(base, EB)],
                                    dstv.at[par])
                    gcp(par).start()

            def drain(i, par):
                @pl.when(s + i * NSUB < HBLK)
                def _():
                    gcp(par).wait()
                    pltpu.sync_copy(gbuf.at[par], acc.at[dstv.at[par]],
                                    add=True)

            start(0, 0)

            def step(j, carry):
                i0 = 2 * j
                start(i0 + 1, 1)
                drain(i0, 0)
                start(i0 + 2, 0)
                drain(i0 + 1, 1)
                return carry

            lax.fori_loop(0, IPS // 2, step, 0)

            plsc.subcore_barrier()
            pltpu.sync_copy(
                acc.at[pl.ds(s * RPS, RPS)],
                out_hbm.at[c, pl.ds(s * RPS, RPS), pl.ds(p * CK, CK)])

    return lap


def _proj_kernel(z_ref, w_ref, b_ref, o_ref):
    o_ref[...] = (jnp.dot(z_ref[...], w_ref[...],
                          preferred_element_type=jnp.float32)
                  + b_ref[...])


def _proj(z, w, b, bn):
    NF = w.shape[1]
    grid = (NF // bn,)
    return pl.pallas_call(
        _proj_kernel,
        grid=grid,
        in_specs=[pl.BlockSpec((B, LATENT), lambda j: (0, 0)),
                  pl.BlockSpec((LATENT, bn), lambda j: (0, j)),
                  pl.BlockSpec((1, bn), lambda j: (0, j))],
        out_specs=pl.BlockSpec((B, bn), lambda j: (0, j)),
        out_shape=jax.ShapeDtypeStruct((B, NF), jnp.float32),
    )(z, w, b)


def _scale_kernel(a_ref, s_ref, o_ref):
    o_ref[...] = a_ref[...] * s_ref[...]


def _scale(a, s, bm=2000):
    M, F = a.shape
    grid = (M // bm,)
    row = lambda i: (i, 0)
    return pl.pallas_call(
        _scale_kernel,
        grid=grid,
        in_specs=[pl.BlockSpec((bm, F), row), pl.BlockSpec((bm, 1), row)],
        out_specs=pl.BlockSpec((bm, F), row),
        out_shape=jax.ShapeDtypeStruct((M, F), jnp.float32),
    )(a, s)


def _elu(x):
    return jnp.where(x > 0.0, x, jnp.exp(x) - 1.0)


def _combine_a_kernel(x_ref, sa_ref, sb_ref, din_ref, dout_ref,
                      w0_ref, w1_ref, w2_ref, b_ref, o_ref, y_ref):
    x = x_ref[...]
    din = din_ref[...]
    t1 = -din * sa_ref[...]
    t2 = -2.0 * din * sb_ref[...] - x
    acc = jnp.dot(x, w0_ref[...], preferred_element_type=jnp.float32)
    acc += jnp.dot(t1, w1_ref[...], preferred_element_type=jnp.float32)
    acc += jnp.dot(t2, w2_ref[...], preferred_element_type=jnp.float32)
    acc = _elu(acc + b_ref[...])
    o_ref[...] = acc
    y_ref[...] = acc * dout_ref[...]


def _combine_a(x, sa, sb, din, dout, W, b, bm):
    M, F = x.shape
    O = W.shape[2]
    grid = (M // bm,)
    full = lambda i: (0, 0)
    row = lambda i: (i, 0)
    return pl.pallas_call(
        _combine_a_kernel,
        grid=grid,
        in_specs=[pl.BlockSpec((bm, F), row), pl.BlockSpec((bm, F), row),
                  pl.BlockSpec((bm, F), row), pl.BlockSpec((bm, 1), row),
                  pl.BlockSpec((bm, 1), row),
                  pl.BlockSpec((F, O), full), pl.BlockSpec((F, O), full),
                  pl.BlockSpec((F, O), full), pl.BlockSpec((1, O), full)],
        out_specs=[pl.BlockSpec((bm, O), row), pl.BlockSpec((bm, O), row)],
        out_shape=[jax.ShapeDtypeStruct((M, O), jnp.float32),
                   jax.ShapeDtypeStruct((M, O), jnp.float32)],
    )(x, sa, sb, din, dout, W[0], W[1], W[2], b)


def _combine_b_kernel(x_ref, sa_ref, sb_ref, din_ref, dout_ref,
                      w0_ref, w1_ref, w2_ref, b_ref, wp1_ref, wp2_ref,
                      o_ref, yq_ref):
    x = x_ref[...]
    din = din_ref[...]
    t1 = -din * sa_ref[...]
    t2 = -2.0 * din * sb_ref[...] - x
    acc = jnp.dot(x, w0_ref[...], preferred_element_type=jnp.float32)
    acc += jnp.dot(t1, w1_ref[...], preferred_element_type=jnp.float32)
    acc += jnp.dot(t2, w2_ref[...], preferred_element_type=jnp.float32)
    acc = _elu(acc + b_ref[...])
    o_ref[...] = acc
    p1 = jnp.dot(acc, wp1_ref[...], preferred_element_type=jnp.float32)
    p2 = jnp.dot(acc, wp2_ref[...], preferred_element_type=jnp.float32)
    yq_ref[...] = jnp.concatenate([p1, p2], axis=1) * dout_ref[...]


def _combine_b(x, sa, sb, din, dout, W, b, wp1, wp2, bm):
    M, F = x.shape
    O = W.shape[2]
    O2 = 2 * wp1.shape[1]
    grid = (M // bm,)
    full = lambda i: (0, 0)
    row = lambda i: (i, 0)
    return pl.pallas_call(
        _combine_b_kernel,
        grid=grid,
        in_specs=[pl.BlockSpec((bm, F), row), pl.BlockSpec((bm, F), row),
                  pl.BlockSpec((bm, F), row), pl.BlockSpec((bm, 1), row),
                  pl.BlockSpec((bm, 1), row),
                  pl.BlockSpec((F, O), full), pl.BlockSpec((F, O), full),
                  pl.BlockSpec((F, O), full), pl.BlockSpec((1, O), full),
                  pl.BlockSpec((O, wp1.shape[1]), full),
                  pl.BlockSpec((O, wp2.shape[1]), full)],
        out_specs=[pl.BlockSpec((bm, O), row), pl.BlockSpec((bm, O2), row)],
        out_shape=[jax.ShapeDtypeStruct((M, O), jnp.float32),
                   jax.ShapeDtypeStruct((M, O2), jnp.float32)],
    )(x, sa, sb, din, dout, W[0], W[1], W[2], b, wp1, wp2)


def _final_kernel(x_ref, u1_ref, u3_ref, din_ref, wd_ref, b_ref, o_ref):
    din = din_ref[...]
    o_ref[...] = (jnp.dot(x_ref[...], wd_ref[...],
                          preferred_element_type=jnp.float32)
                  - din * u1_ref[...] - 2.0 * din * u3_ref[...] + b_ref[...])


def _final(x, u1, u3, din, wd, b, bm):
    M, F = x.shape
    O = wd.shape[1]
    grid = (M // bm,)
    full = lambda i: (0, 0)
    row = lambda i: (i, 0)
    return pl.pallas_call(
        _final_kernel,
        grid=grid,
        in_specs=[pl.BlockSpec((bm, F), row), pl.BlockSpec((bm, O), row),
                  pl.BlockSpec((bm, O), row), pl.BlockSpec((bm, 1), row),
                  pl.BlockSpec((F, O), full), pl.BlockSpec((1, O), full)],
        out_specs=pl.BlockSpec((bm, O), row),
        out_shape=jax.ShapeDtypeStruct((M, O), jnp.float32),
    )(x, u1, u3, din, wd, b)


def kernel(z, edge_index, proj_W, proj_b, W0, b0, W1, b1, W2, b2):
    src = edge_index[0].astype(jnp.int32)
    dst = edge_index[1].astype(jnp.int32)
    deg_out = jnp.zeros((N,), jnp.float32).at[src].add(1.0)
    deg_in = jnp.zeros((N,), jnp.float32).at[dst].add(1.0)
    dinv_out = lax.rsqrt(jnp.maximum(deg_out, 1.0))
    dinv_in = lax.rsqrt(jnp.maximum(deg_in, 1.0))
    ms = -(dinv_out * dinv_in)

    din_col = jnp.repeat(dinv_in, B)[:, None]
    dout_col = jnp.repeat(dinv_out, B)[:, None]
    ms_col = jnp.repeat(ms, B)[:, None]
    zbuf = jnp.zeros((RPS, CK), jnp.float32)

    lap512 = _make_lap(512)
    lap1024 = _make_lap(1024)
    lap256 = _make_lap(256)
    M = N * B

    x0 = _proj(z, proj_W, proj_b.reshape(1, -1), 2560)
    xt = jnp.swapaxes(x0.reshape(B, N, 32), 0, 1)
    x0r = xt.reshape(M, 32)
    y0 = _scale(x0r, dout_col)
    sa2 = lap512(y0.reshape(N, 512), src, dst, zbuf)
    sa = sa2[0] + sa2[1]
    san = sa[:N].reshape(M, 32)
    y1 = _scale(san, ms_col)
    sb2 = lap512(y1.reshape(N, 512), src, dst, zbuf)
    sb = sb2[0] + sb2[1]
    sbn = sb[:N].reshape(M, 32)
    out0, ynext = _combine_a(x0r, san, sbn, din_col, dout_col,
                             W0, b0.reshape(1, -1), 1000)

    sa12 = lap1024(ynext.reshape(N, 1024), src, dst, zbuf)
    sa1 = sa12[0] + sa12[1]
    sa1n = sa1[:N].reshape(M, 64)
    y11 = _scale(sa1n, ms_col)
    sb12 = lap1024(y11.reshape(N, 1024), src, dst, zbuf)
    sb1 = sb12[0] + sb12[1]
    sb1n = sb1[:N].reshape(M, 64)
    out1, yq = _combine_b(out0, sa1n, sb1n, din_col, dout_col,
                          W1, b1.reshape(1, -1), W2[1], W2[2], 1000)

    u122 = lap512(yq.reshape(N, 512), src, dst, zbuf)
    u12 = u122[0] + u122[1]
    u12n = u12[:N].reshape(M, 32)
    u1 = u12n[:, :16]
    u2 = u12n[:, 16:]
    yq3 = _scale(u2, ms_col)
    u3p2 = lap256(yq3.reshape(N, 256), src, dst, zbuf)
    u3p = u3p2[0] + u3p2[1]
    u3 = u3p[:N].reshape(M, 16)
    out2 = _final(out1, u1, u3, din_col, W2[0] - W2[2],
                  b2.reshape(1, -1), 1000)
    return jnp.swapaxes(out2.reshape(N, B, 16), 0, 1)
